# Initial kernel scaffold; baseline (speedup 1.0000x reference)
import jax
import jax.numpy as jnp
from jax.experimental import pallas as pl


def kernel(node_x, edge_index, edge_z, W_init, b_init, W1, b1, W2, b2, W_m2m, b_m2m, W_inj, b_inj, W_ro, b_ro, W_head, b_head):
    raise NotImplementedError("write your pallas kernel here")


# speedup vs baseline: 3.3161x; 3.3161x over previous
import functools

import jax
import jax.numpy as jnp
from jax import lax
from jax.experimental import pallas as pl
from jax.experimental.pallas import tpu as pltpu
from jax.experimental.pallas import tpu_sc as plsc

NC = 2
NS = 16
LANES = 16

D = 128
K = 80


def _node_dense_body(x_ref, wi_ref, bi_ref, w1a_ref, w1b_ref, wj_ref, bj_ref,
                     mem_ref, a_ref, b_ref, inj_ref):
    x = x_ref[...]
    mem = jnp.tanh(
        jnp.dot(x, wi_ref[...], preferred_element_type=jnp.float32)
        + bi_ref[...])
    mem_ref[...] = mem
    a_ref[...] = jnp.dot(mem, w1a_ref[...], preferred_element_type=jnp.float32)
    b_ref[...] = jnp.dot(mem, w1b_ref[...], preferred_element_type=jnp.float32)
    inj_ref[...] = 0.1 * jnp.tanh(
        jnp.dot(x, wj_ref[...], preferred_element_type=jnp.float32)
        + bj_ref[...])


def _edge_dense_body(z_ref, w1c_ref, b1_ref, c_ref):
    c_ref[...] = (
        jnp.dot(z_ref[...], w1c_ref[...], preferred_element_type=jnp.float32)
        + b1_ref[...])


def _sc_edge_body(n_pad, e_per_w, a_hbm, b_hbm, c_hbm, src_hbm, dst_hbm,
                  out_hbm, cnt_hbm, srcv, dstv, av, bv, cv, rsh,
                  sema, semb, semc):
    cid = lax.axis_index("c")
    sid = lax.axis_index("s")
    wid = cid * NS + sid
    rows_per_tile = n_pad // NS
    row0 = sid * rows_per_tile
    ew_base = wid * e_per_w

    zeros16 = jnp.zeros((LANES,), jnp.float32)
    ones16 = jnp.ones((LANES,), jnp.float32)

    def _fill(ref, val16):
        def _row(r, _):
            for j in range(D // LANES):
                ref[r, pl.ds(j * LANES, LANES)] = val16
            return _
        lax.fori_loop(0, K, _row, None)

    _fill(bv, zeros16)
    for t in range(rows_per_tile // K):
        pltpu.sync_copy(bv, rsh.at[pl.ds(row0 + t * K, K)])

    plsc.subcore_barrier()

    def _chunk(it, _):
        base = ew_base + it * K
        pltpu.sync_copy(src_hbm.at[pl.ds(base, K)], srcv)
        pltpu.sync_copy(dst_hbm.at[pl.ds(base, K)], dstv)
        cpa = pltpu.async_copy(a_hbm.at[srcv], av, sema)
        cpb = pltpu.async_copy(b_hbm.at[dstv], bv, semb)
        cpc = pltpu.async_copy(c_hbm.at[pl.ds(base, K)], cv, semc)
        cpa.wait()
        cpb.wait()
        cpc.wait()

        def _row(r, _):
            for j in range(D // LANES):
                sl = pl.ds(j * LANES, LANES)
                v = av[r, sl] + bv[r, sl] + cv[r, sl]
                av[r, sl] = jnp.maximum(v, 0.0)
            return _
        lax.fori_loop(0, K, _row, None)

        pltpu.sync_copy(av, rsh.at[dstv], add=True)
        return _

    lax.fori_loop(0, e_per_w // K, _chunk, None)

    plsc.subcore_barrier()

    pltpu.sync_copy(rsh.at[pl.ds(row0, rows_per_tile)],
                    out_hbm.at[cid, pl.ds(row0, rows_per_tile)])

    plsc.subcore_barrier()

    _fill(av, zeros16)
    for t in range(rows_per_tile // K):
        pltpu.sync_copy(av, rsh.at[pl.ds(row0 + t * K, K)])
    _fill(bv, ones16)

    plsc.subcore_barrier()

    def _cchunk(it, _):
        base = ew_base + it * K
        pltpu.sync_copy(dst_hbm.at[pl.ds(base, K)], dstv)
        pltpu.sync_copy(bv, rsh.at[dstv], add=True)
        return _

    lax.fori_loop(0, e_per_w // K, _cchunk, None)

    plsc.subcore_barrier()

    pltpu.sync_copy(rsh.at[pl.ds(row0, rows_per_tile)],
                    cnt_hbm.at[cid, pl.ds(row0, rows_per_tile)])


def _post_body(n, rparts_ref, cparts_ref, mem_ref, inj_ref, w2_ref, b2_ref,
               wm_ref, bm_ref, wro_ref, bro_ref, wh_ref, bh_ref, pred_ref):
    rsum = rparts_ref[0, :n, :] + rparts_ref[1, :n, :]
    cnt = cparts_ref[0, :n, 0] + cparts_ref[1, :n, 0]
    agg = (jnp.dot(rsum, w2_ref[...], preferred_element_type=jnp.float32)
           + cnt[:, None] * b2_ref[...])
    agg_mem = (jnp.dot(agg, wm_ref[...], preferred_element_type=jnp.float32)
               + bm_ref[...])
    new_mem = 0.9 * jnp.tanh(mem_ref[...] + agg_mem) + inj_ref[...]
    emb = jax.nn.relu(
        jnp.dot(new_mem, wro_ref[...], preferred_element_type=jnp.float32)
        + bro_ref[...])
    pred_ref[...] = (
        jnp.dot(emb, wh_ref[...], preferred_element_type=jnp.float32)
        + bh_ref[...])


def kernel(node_x, edge_index, edge_z, W_init, b_init, W1, b1, W2, b2,
           W_m2m, b_m2m, W_inj, b_inj, W_ro, b_ro, W_head, b_head):
    n, node_in = node_x.shape
    e, edge_in = edge_z.shape
    mem_w = W_init.shape[1]
    assert mem_w == D and W1.shape[1] == D

    W1a = W1[:mem_w]
    W1b = W1[mem_w:2 * mem_w]
    W1c = W1[2 * mem_w:]
    src = edge_index[0].astype(jnp.int32)
    dst = edge_index[1].astype(jnp.int32)

    f32 = jnp.float32
    mem, a_tab, b_tab, inj = pl.pallas_call(
        _node_dense_body,
        out_shape=[jax.ShapeDtypeStruct((n, D), f32)] * 4,
    )(node_x, W_init, b_init.reshape(1, D), W1a, W1b, W_inj,
      b_inj.reshape(1, D))

    eb = 20000
    assert e % eb == 0
    c_tab = pl.pallas_call(
        _edge_dense_body,
        grid=(e // eb,),
        in_specs=[
            pl.BlockSpec((eb, edge_in), lambda i: (i, 0)),
            pl.BlockSpec((edge_in, D), lambda i: (0, 0)),
            pl.BlockSpec((1, D), lambda i: (0, 0)),
        ],
        out_specs=pl.BlockSpec((eb, D), lambda i: (i, 0)),
        out_shape=jax.ShapeDtypeStruct((e, D), f32),
    )(edge_z, W1c, b1.reshape(1, D))

    nw = NC * NS
    e_per_w = e // nw
    assert e % nw == 0 and e_per_w % K == 0
    n_pad = ((n + NS * K - 1) // (NS * K)) * (NS * K)

    rparts, cparts = pl.kernel(
        functools.partial(_sc_edge_body, n_pad, e_per_w),
        out_type=[jax.ShapeDtypeStruct((NC, n_pad, D), f32),
                  jax.ShapeDtypeStruct((NC, n_pad, D), f32)],
        mesh=plsc.VectorSubcoreMesh(
            core_axis_name="c", subcore_axis_name="s",
            num_cores=NC, num_subcores=NS),
        scratch_types=[
            pltpu.VMEM((K,), jnp.int32),
            pltpu.VMEM((K,), jnp.int32),
            pltpu.VMEM((K, D), f32),
            pltpu.VMEM((K, D), f32),
            pltpu.VMEM((K, D), f32),
            pltpu.VMEM_SHARED((n_pad, D), f32),
            pltpu.SemaphoreType.DMA,
            pltpu.SemaphoreType.DMA,
            pltpu.SemaphoreType.DMA,
        ],
    )(a_tab, b_tab, c_tab, src, dst)

    pred = pl.pallas_call(
        functools.partial(_post_body, n),
        out_shape=jax.ShapeDtypeStruct((n, 1), f32),
    )(rparts, cparts, mem, inj, W2, b2.reshape(1, D), W_m2m, b_m2m.reshape(1, D),
      W_ro, b_ro.reshape(1, D), W_head, b_head.reshape(1, 1))

    return pred
